# baseline (speedup 1.0000x reference)
import functools

import jax
import jax.numpy as jnp
from jax.experimental import pallas as pl
from jax.experimental.pallas import tpu as pltpu


def _chamfer_body(a_ref, b_ref, acc_ref, dist1_s, dist2_s, *, inv_bn, inv_bm):
    n = pl.program_id(1)
    m = pl.program_id(2)

    a = a_ref[0]
    bt = b_ref[0]
    x0 = a[:, 0:1]
    x1 = a[:, 1:2]
    x2 = a[:, 2:3]
    xsq = a[:, 3:4]
    y0 = bt[0:1, :]
    y1 = bt[1:2, :]
    y2 = bt[2:3, :]
    ysq = bt[4:5, :]
    d = (xsq + ysq) + (x0 * y0 + x1 * y1 + x2 * y2)

    rowmin = jnp.min(d, axis=1)
    colmin = jnp.min(d, axis=0)

    prev1 = jnp.where(m == 0, jnp.inf, dist1_s[...])
    new1 = jnp.minimum(prev1, rowmin)
    dist1_s[...] = new1

    prev2 = jnp.where(n == 0, jnp.inf, dist2_s[m])
    new2 = jnp.minimum(prev2, colmin)
    dist2_s[m] = new2

    @pl.when((pl.program_id(0) == 0) & (n == 0) & (m == 0))
    def _init():
        acc_ref[0, 0] = 0.0

    @pl.when(m == pl.num_programs(2) - 1)
    def _fin1():
        acc_ref[0, 0] += jnp.sum(new1) * inv_bn

    @pl.when(n == pl.num_programs(1) - 1)
    def _fin2():
        acc_ref[0, 0] += jnp.sum(new2) * inv_bm


@jax.jit
def kernel(xyz1, xyz2):
    B, N, _ = xyz1.shape
    M = xyz2.shape[1]
    BN = 256
    BM = 1024
    NB = N // BN
    MB = M // BM

    xsq = jnp.sum(xyz1 * xyz1, axis=-1, keepdims=True)
    ysq = jnp.sum(xyz2 * xyz2, axis=-1, keepdims=True)
    ones_x = jnp.ones((B, N, 1), jnp.float32)
    ones_y = jnp.ones((B, M, 1), jnp.float32)
    zeros_x = jnp.zeros((B, N, 3), jnp.float32)
    zeros_y = jnp.zeros((B, M, 3), jnp.float32)
    xr = xyz1.astype(jnp.bfloat16).astype(jnp.float32)
    yr = xyz2.astype(jnp.bfloat16).astype(jnp.float32)
    a = jnp.concatenate([xr, xsq, ones_x, zeros_x], axis=-1)
    bt = jnp.concatenate([-2.0 * yr, ones_y, ysq, zeros_y], axis=-1)
    bt = jnp.transpose(bt, (0, 2, 1))

    body = functools.partial(
        _chamfer_body, inv_bn=1.0 / (B * N), inv_bm=1.0 / (B * M)
    )
    acc = pl.pallas_call(
        body,
        grid=(B, NB, MB),
        in_specs=[
            pl.BlockSpec((1, BN, 8), lambda b, n, m: (b, n, 0)),
            pl.BlockSpec((1, 8, BM), lambda b, n, m: (b, 0, m)),
        ],
        out_specs=pl.BlockSpec(
            (1, 1), lambda b, n, m: (0, 0), memory_space=pltpu.SMEM
        ),
        out_shape=jax.ShapeDtypeStruct((1, 1), jnp.float32),
        scratch_shapes=[
            pltpu.VMEM((BN,), jnp.float32),
            pltpu.VMEM((MB, BM), jnp.float32),
        ],
    )(a, bt)
    return acc[0, 0]


# speedup vs baseline: 1.0498x; 1.0498x over previous
import functools

import jax
import jax.numpy as jnp
from jax.experimental import pallas as pl
from jax.experimental.pallas import tpu as pltpu


def _chamfer_body(
    x_ref, yt_ref, xsq_ref, ysq_ref, acc_ref, dist1_s, dist2_s, *, inv_bn, inv_bm
):
    n = pl.program_id(1)
    m = pl.program_id(2)

    x = x_ref[0]
    yt = yt_ref[0]
    inner = jnp.dot(x, yt, preferred_element_type=jnp.float32)
    e = ysq_ref[0] - 2.0 * inner
    d = e + xsq_ref[0]

    rowmin = jnp.min(d, axis=1)
    colmin = jnp.min(d, axis=0)

    prev1 = jnp.where(m == 0, jnp.inf, dist1_s[...])
    new1 = jnp.minimum(prev1, rowmin)
    dist1_s[...] = new1

    prev2 = jnp.where(n == 0, jnp.inf, dist2_s[m])
    new2 = jnp.minimum(prev2, colmin)
    dist2_s[m] = new2

    @pl.when((pl.program_id(0) == 0) & (n == 0) & (m == 0))
    def _init():
        acc_ref[0, 0] = 0.0

    @pl.when(m == pl.num_programs(2) - 1)
    def _fin1():
        acc_ref[0, 0] += jnp.sum(new1) * inv_bn

    @pl.when(n == pl.num_programs(1) - 1)
    def _fin2():
        acc_ref[0, 0] += jnp.sum(new2) * inv_bm


@jax.jit
def kernel(xyz1, xyz2):
    B, N, _ = xyz1.shape
    M = xyz2.shape[1]
    BN = 256
    BM = 1024
    NB = N // BN
    MB = M // BM

    xsq = jnp.sum(xyz1 * xyz1, axis=-1, keepdims=True)
    ysq = jnp.sum(xyz2 * xyz2, axis=-1)[:, None, :]
    pad_x = jnp.zeros((B, N, 5), jnp.bfloat16)
    pad_y = jnp.zeros((B, M, 5), jnp.bfloat16)
    xb = jnp.concatenate([xyz1.astype(jnp.bfloat16), pad_x], axis=-1)
    yb = jnp.concatenate([xyz2.astype(jnp.bfloat16), pad_y], axis=-1)
    ybt = jnp.transpose(yb, (0, 2, 1))

    body = functools.partial(
        _chamfer_body, inv_bn=1.0 / (B * N), inv_bm=1.0 / (B * M)
    )
    acc = pl.pallas_call(
        body,
        grid=(B, NB, MB),
        in_specs=[
            pl.BlockSpec((1, BN, 8), lambda b, n, m: (b, n, 0)),
            pl.BlockSpec((1, 8, BM), lambda b, n, m: (b, 0, m)),
            pl.BlockSpec((1, BN, 1), lambda b, n, m: (b, n, 0)),
            pl.BlockSpec((1, 1, BM), lambda b, n, m: (b, 0, m)),
        ],
        out_specs=pl.BlockSpec(
            (1, 1), lambda b, n, m: (0, 0), memory_space=pltpu.SMEM
        ),
        out_shape=jax.ShapeDtypeStruct((1, 1), jnp.float32),
        scratch_shapes=[
            pltpu.VMEM((BN,), jnp.float32),
            pltpu.VMEM((MB, BM), jnp.float32),
        ],
    )(xb, ybt, xsq, ysq)
    return acc[0, 0]


# speedup vs baseline: 1.9150x; 1.8242x over previous
import functools

import jax
import jax.numpy as jnp
from jax.experimental import pallas as pl
from jax.experimental.pallas import tpu as pltpu


def _tree_min(chunks):
    while len(chunks) > 1:
        nxt = []
        for i in range(0, len(chunks) - 1, 2):
            nxt.append(jnp.minimum(chunks[i], chunks[i + 1]))
        if len(chunks) % 2:
            nxt.append(chunks[-1])
        chunks = nxt
    return chunks[0]


def _split3(v):
    h1 = v.astype(jnp.bfloat16)
    r1 = v - h1.astype(jnp.float32)
    h2 = r1.astype(jnp.bfloat16)
    r2 = r1 - h2.astype(jnp.float32)
    h3 = r2.astype(jnp.bfloat16)
    return h1, h2, h3


def _chamfer_body(
    a_ref, bt_ref, acc_ref, row_s, col_s, *, bn, bm, inv_bn, inv_bm
):
    n = pl.program_id(1)
    m = pl.program_id(2)

    a = a_ref[0]
    bt = bt_ref[0]
    d = jnp.dot(a, bt, preferred_element_type=jnp.float32)

    rowpart = _tree_min([d[:, k : k + 128] for k in range(0, bm, 128)])
    prev1 = jnp.where(m == 0, jnp.inf, row_s[...])
    new1 = jnp.minimum(prev1, rowpart)
    row_s[...] = new1

    colpart = _tree_min([d[k : k + 8, :] for k in range(0, bn, 8)])
    prev2 = jnp.where(n == 0, jnp.inf, col_s[m])
    new2 = jnp.minimum(prev2, colpart)
    col_s[m] = new2

    @pl.when((pl.program_id(0) == 0) & (n == 0) & (m == 0))
    def _init():
        acc_ref[0, 0] = 0.0

    @pl.when(m == pl.num_programs(2) - 1)
    def _fin1():
        acc_ref[0, 0] += jnp.sum(jnp.min(new1, axis=1)) * inv_bn

    @pl.when(n == pl.num_programs(1) - 1)
    def _fin2():
        acc_ref[0, 0] += jnp.sum(jnp.min(new2, axis=0)) * inv_bm


@jax.jit
def kernel(xyz1, xyz2):
    B, N, _ = xyz1.shape
    M = xyz2.shape[1]
    BN = 256
    BM = 2048
    NB = N // BN
    MB = M // BM

    xsq = jnp.sum(xyz1 * xyz1, axis=-1)
    ysq = jnp.sum(xyz2 * xyz2, axis=-1)
    xh1, xh2, xh3 = _split3(xsq)
    yh1, yh2, yh3 = _split3(ysq)

    xb = xyz1.astype(jnp.bfloat16)
    yb = xyz2.astype(jnp.bfloat16)
    ones_x = jnp.ones((B, N, 3), jnp.bfloat16)
    ones_y = jnp.ones((B, M, 3), jnp.bfloat16)
    zeros_x = jnp.zeros((B, N, 7), jnp.bfloat16)
    zeros_y = jnp.zeros((B, M, 7), jnp.bfloat16)
    stack_x = jnp.stack([xh1, xh2, xh3], axis=-1)
    stack_y = jnp.stack([yh1, yh2, yh3], axis=-1)
    a = jnp.concatenate([xb, stack_x, ones_x, zeros_x], axis=-1)
    bt = jnp.concatenate([-2.0 * yb, ones_y, stack_y, zeros_y], axis=-1)
    bt = jnp.transpose(bt, (0, 2, 1))

    body = functools.partial(
        _chamfer_body, bn=BN, bm=BM, inv_bn=1.0 / (B * N), inv_bm=1.0 / (B * M)
    )
    acc = pl.pallas_call(
        body,
        grid=(B, NB, MB),
        in_specs=[
            pl.BlockSpec((1, BN, 16), lambda b, n, m: (b, n, 0)),
            pl.BlockSpec((1, 16, BM), lambda b, n, m: (b, 0, m)),
        ],
        out_specs=pl.BlockSpec(
            (1, 1), lambda b, n, m: (0, 0), memory_space=pltpu.SMEM
        ),
        out_shape=jax.ShapeDtypeStruct((1, 1), jnp.float32),
        scratch_shapes=[
            pltpu.VMEM((BN, 128), jnp.float32),
            pltpu.VMEM((MB, 8, BM), jnp.float32),
        ],
    )(a, bt)
    return acc[0, 0]
